# baseline (device time: 21633 ns/iter reference)
import os

import jax
import jax.numpy as jnp
from jax import lax
from jax.experimental import pallas as pl
from jax.experimental.pallas import tpu as pltpu

N_DEV = 8
BLOCK_M = int(os.environ.get("KBLOCK_M", "2048"))
NCHUNKS = int(os.environ.get("KNCHUNKS", "2"))
_LOCAL_ONLY = os.environ.get("LOCAL_ONLY") == "1"


def kernel(x):
    m_per, n = x.shape
    nblocks = m_per // BLOCK_M
    w = n // NCHUNKS

    def _rdma(c, k, me, pbuf, rbuf, ssems, rsems):
        return pltpu.make_async_remote_copy(
            src_ref=pbuf.at[c],
            dst_ref=rbuf.at[c, k - 1],
            send_sem=ssems.at[c, k - 1],
            recv_sem=rsems.at[c, k - 1],
            device_id=((me + k) % N_DEV,),
            device_id_type=pl.DeviceIdType.MESH,
        )

    def body(x_ref, o_ref, pbuf, rbuf, ssems, rsems):
        c = pl.program_id(0)
        r = pl.program_id(1)
        me = lax.axis_index("i")

        xb = x_ref[...]
        m = jnp.max(xb, axis=0, keepdims=True)
        rows = lax.broadcasted_iota(jnp.int32, xb.shape, 0)
        li = jnp.min(
            jnp.where(xb == m, rows, BLOCK_M), axis=0, keepdims=True
        )
        gidx = li.astype(jnp.float32) + (
            me.astype(jnp.float32) * float(m_per)
            + r.astype(jnp.float32) * float(BLOCK_M)
        )

        @pl.when(r == 0)
        def _():
            pbuf[c, 0:1, :] = m
            pbuf[c, 1:2, :] = gidx

        @pl.when(r > 0)
        def _():
            take = m > pbuf[c, 0:1, :]
            pbuf[c, 0:1, :] = jnp.where(take, m, pbuf[c, 0:1, :])
            pbuf[c, 1:2, :] = jnp.where(take, gidx, pbuf[c, 1:2, :])

        if _LOCAL_ONLY:
            @pl.when(
                jnp.logical_and(c == NCHUNKS - 1, r == nblocks - 1)
            )
            def _():
                for cc in range(NCHUNKS):
                    o_ref[:, cc * w:(cc + 1) * w] = pbuf[cc]
            return

        @pl.when(jnp.logical_and(c == 0, r == nblocks - 2))
        def _():
            barrier = pltpu.get_barrier_semaphore()
            for k in range(1, N_DEV):
                pl.semaphore_signal(
                    barrier, inc=1, device_id=((me + k) % N_DEV,),
                    device_id_type=pl.DeviceIdType.MESH,
                )

        @pl.when(jnp.logical_and(c == 0, r == nblocks - 1))
        def _():
            barrier = pltpu.get_barrier_semaphore()
            pl.semaphore_wait(barrier, N_DEV - 1)

        @pl.when(r == nblocks - 1)
        def _():
            for k in range(1, N_DEV):
                _rdma(c, k, me, pbuf, rbuf, ssems, rsems).start()

        @pl.when(jnp.logical_and(c == NCHUNKS - 1, r == nblocks - 1))
        def _():
            for cc in range(NCHUNKS):
                for k in range(1, N_DEV):
                    _rdma(cc, k, me, pbuf, rbuf, ssems, rsems).wait()
            for cc in range(NCHUNKS):
                acc_v = pbuf[cc, 0:1, :]
                acc_i = pbuf[cc, 1:2, :]
                for k in range(1, N_DEV):
                    v = rbuf[cc, k - 1, 0:1, :]
                    ix = rbuf[cc, k - 1, 1:2, :]
                    take = (v > acc_v) | ((v == acc_v) & (ix < acc_i))
                    acc_v = jnp.where(take, v, acc_v)
                    acc_i = jnp.where(take, ix, acc_i)
                o_ref[0:1, cc * w:(cc + 1) * w] = acc_v
                o_ref[1:2, cc * w:(cc + 1) * w] = acc_i

    return pl.pallas_call(
        body,
        grid=(NCHUNKS, nblocks),
        in_specs=[pl.BlockSpec((BLOCK_M, w), lambda c, r: (r, c))],
        out_specs=pl.BlockSpec((2, n), lambda c, r: (0, 0)),
        out_shape=jax.ShapeDtypeStruct((2, n), jnp.float32),
        scratch_shapes=[
            pltpu.VMEM((NCHUNKS, 2, w), jnp.float32),
            pltpu.VMEM((NCHUNKS, N_DEV - 1, 2, w), jnp.float32),
            pltpu.SemaphoreType.DMA((NCHUNKS, N_DEV - 1)),
            pltpu.SemaphoreType.DMA((NCHUNKS, N_DEV - 1)),
        ],
        compiler_params=(
            pltpu.CompilerParams(vmem_limit_bytes=100 * 2**20)
            if _LOCAL_ONLY
            else pltpu.CompilerParams(
                collective_id=0, vmem_limit_bytes=100 * 2**20
            )
        ),
    )(x)


# device time: 20833 ns/iter; 1.0384x vs baseline; 1.0384x over previous
import os

import jax
import jax.numpy as jnp
from jax import lax
from jax.experimental import pallas as pl
from jax.experimental.pallas import tpu as pltpu

N_DEV = 8
BLOCK_M = int(os.environ.get("KBLOCK_M", "2048"))
NCHUNKS = int(os.environ.get("KNCHUNKS", "1"))
_LOCAL_ONLY = os.environ.get("LOCAL_ONLY") == "1"


def kernel(x):
    m_per, n = x.shape
    nblocks = m_per // BLOCK_M
    w = n // NCHUNKS

    def _rdma(c, k, me, pbuf, rbuf, ssems, rsems):
        return pltpu.make_async_remote_copy(
            src_ref=pbuf.at[c],
            dst_ref=rbuf.at[c, k - 1],
            send_sem=ssems.at[c, k - 1],
            recv_sem=rsems.at[c, k - 1],
            device_id=((me + k) % N_DEV,),
            device_id_type=pl.DeviceIdType.MESH,
        )

    def body(x_ref, o_ref, pbuf, rbuf, ssems, rsems):
        c = pl.program_id(0)
        r = pl.program_id(1)
        me = lax.axis_index("i")

        xb = x_ref[...]
        m = jnp.max(xb, axis=0, keepdims=True)
        rows = lax.broadcasted_iota(jnp.int32, xb.shape, 0)
        li = jnp.min(
            jnp.where(xb == m, rows, BLOCK_M), axis=0, keepdims=True
        )
        gidx = li.astype(jnp.float32) + (
            me.astype(jnp.float32) * float(m_per)
            + r.astype(jnp.float32) * float(BLOCK_M)
        )

        @pl.when(r == 0)
        def _():
            pbuf[c, 0:1, :] = m
            pbuf[c, 1:2, :] = gidx

        @pl.when(r > 0)
        def _():
            take = m > pbuf[c, 0:1, :]
            pbuf[c, 0:1, :] = jnp.where(take, m, pbuf[c, 0:1, :])
            pbuf[c, 1:2, :] = jnp.where(take, gidx, pbuf[c, 1:2, :])

        if _LOCAL_ONLY:
            @pl.when(
                jnp.logical_and(c == NCHUNKS - 1, r == nblocks - 1)
            )
            def _():
                for cc in range(NCHUNKS):
                    o_ref[:, cc * w:(cc + 1) * w] = pbuf[cc]
            return

        @pl.when(jnp.logical_and(c == 0, r == nblocks - 2))
        def _():
            barrier = pltpu.get_barrier_semaphore()
            for k in range(1, N_DEV):
                pl.semaphore_signal(
                    barrier, inc=1, device_id=((me + k) % N_DEV,),
                    device_id_type=pl.DeviceIdType.MESH,
                )

        @pl.when(jnp.logical_and(c == 0, r == nblocks - 1))
        def _():
            barrier = pltpu.get_barrier_semaphore()
            pl.semaphore_wait(barrier, N_DEV - 1)

        @pl.when(r == nblocks - 1)
        def _():
            for k in range(1, N_DEV):
                _rdma(c, k, me, pbuf, rbuf, ssems, rsems).start()

        @pl.when(jnp.logical_and(c == NCHUNKS - 1, r == nblocks - 1))
        def _():
            for cc in range(NCHUNKS):
                for k in range(1, N_DEV):
                    _rdma(cc, k, me, pbuf, rbuf, ssems, rsems).wait()
            for cc in range(NCHUNKS):
                acc_v = pbuf[cc, 0:1, :]
                acc_i = pbuf[cc, 1:2, :]
                for k in range(1, N_DEV):
                    v = rbuf[cc, k - 1, 0:1, :]
                    ix = rbuf[cc, k - 1, 1:2, :]
                    take = (v > acc_v) | ((v == acc_v) & (ix < acc_i))
                    acc_v = jnp.where(take, v, acc_v)
                    acc_i = jnp.where(take, ix, acc_i)
                o_ref[0:1, cc * w:(cc + 1) * w] = acc_v
                o_ref[1:2, cc * w:(cc + 1) * w] = acc_i

    return pl.pallas_call(
        body,
        grid=(NCHUNKS, nblocks),
        in_specs=[pl.BlockSpec((BLOCK_M, w), lambda c, r: (r, c))],
        out_specs=pl.BlockSpec((2, n), lambda c, r: (0, 0)),
        out_shape=jax.ShapeDtypeStruct((2, n), jnp.float32),
        scratch_shapes=[
            pltpu.VMEM((NCHUNKS, 2, w), jnp.float32),
            pltpu.VMEM((NCHUNKS, N_DEV - 1, 2, w), jnp.float32),
            pltpu.SemaphoreType.DMA((NCHUNKS, N_DEV - 1)),
            pltpu.SemaphoreType.DMA((NCHUNKS, N_DEV - 1)),
        ],
        compiler_params=(
            pltpu.CompilerParams(vmem_limit_bytes=100 * 2**20)
            if _LOCAL_ONLY
            else pltpu.CompilerParams(
                collective_id=0, vmem_limit_bytes=100 * 2**20
            )
        ),
    )(x)
